# Initial kernel scaffold; baseline (speedup 1.0000x reference)
import jax
import jax.numpy as jnp
from jax.experimental import pallas as pl


def kernel(x, edge_index, batch, W1a, b1a, W1b, b1b, W2a, b2a, W2b, b2b, Wfc, bfc):
    raise NotImplementedError("write your pallas kernel here")


# speedup vs baseline: 8.1902x; 8.1902x over previous
import functools

import jax
import jax.numpy as jnp
from jax import lax
from jax.experimental import pallas as pl
from jax.experimental.pallas import tpu as pltpu
from jax.experimental.pallas import tpu_sc as plsc

N_NODES = 10000
N_EDGES = 320000
HIDDEN = 128
N_GRAPHS = 64

NC = 2
NS = 16
NW = NC * NS
CHUNK = 128
CH = 79
EPW = CH * CHUNK
NP = 10240
ROWS_PER_TILE = NP // NS


def _make_sc_agg(n_rows: int):
    mesh = plsc.VectorSubcoreMesh(core_axis_name="c", subcore_axis_name="s")

    @functools.partial(
        pl.kernel,
        mesh=mesh,
        out_type=jax.ShapeDtypeStruct((NC, NP, HIDDEN), jnp.float32),
        scratch_types=[
            pltpu.VMEM((CH, CHUNK), jnp.int32),
            pltpu.VMEM((CH, CHUNK), jnp.int32),
            pltpu.VMEM((CHUNK, HIDDEN), jnp.float32),
            pltpu.VMEM_SHARED((NP, HIDDEN), jnp.float32),
            pltpu.SemaphoreType.DMA,
        ],
    )
    def k(x_hbm, src_hbm, dst_hbm, out_hbm, src_v, dst_v, rows_v, acc, sem):
        c = lax.axis_index("c")
        s = lax.axis_index("s")
        w = c * NS + s

        def zrow(i, carry):
            for j in range(HIDDEN // 16):
                rows_v[i, pl.ds(j * 16, 16)] = jnp.zeros((16,), jnp.float32)
            return carry

        lax.fori_loop(0, CHUNK, zrow, 0)
        for kk in range(ROWS_PER_TILE // CHUNK):
            pltpu.sync_copy(rows_v, acc.at[pl.ds(s * ROWS_PER_TILE + kk * CHUNK, CHUNK)])

        pltpu.sync_copy(src_hbm.at[w], src_v)
        pltpu.sync_copy(dst_hbm.at[w], dst_v)
        plsc.subcore_barrier()

        def body(j, carry):
            pltpu.async_copy(x_hbm.at[src_v.at[j]], rows_v, sem).wait()
            pltpu.sync_copy(rows_v, acc.at[dst_v.at[j]], add=True)
            return carry

        lax.fori_loop(0, CH, body, 0)
        plsc.subcore_barrier()

        pltpu.sync_copy(
            acc.at[pl.ds(s * ROWS_PER_TILE, ROWS_PER_TILE)],
            out_hbm.at[c, pl.ds(s * ROWS_PER_TILE, ROWS_PER_TILE)],
        )

    return k


_INV_SQRT2 = 0.7071067811865476


def _gelu(z):
    return 0.5 * z * (1.0 + lax.erf(z * _INV_SQRT2))


def _mlp1_body(x_ref, a0_ref, a1_ref, wa_ref, ba_ref, wb_ref, bb_ref, o_ref):
    t = x_ref[...] + a0_ref[...] + a1_ref[...]
    z = jnp.dot(t, wa_ref[...], preferred_element_type=jnp.float32) + ba_ref[...]
    z = _gelu(z)
    z = jnp.dot(z, wb_ref[...], preferred_element_type=jnp.float32) + bb_ref[...]
    o_ref[...] = _gelu(z)


def _mlp2_body(x_ref, a0_ref, a1_ref, wa_ref, ba_ref, wb_ref, bb_ref,
               wfc_ref, bfc_ref, b_ref, o_ref):
    t = x_ref[...] + a0_ref[...] + a1_ref[...]
    z = jnp.dot(t, wa_ref[...], preferred_element_type=jnp.float32) + ba_ref[...]
    z = _gelu(z)
    z = jnp.dot(z, wb_ref[...], preferred_element_type=jnp.float32) + bb_ref[...]
    z = _gelu(z)
    v = jnp.dot(z, wfc_ref[...], preferred_element_type=jnp.float32) + bfc_ref[...]
    bb = b_ref[0, 0, :]
    onehot = (bb[None, :] == lax.broadcasted_iota(jnp.int32, (N_GRAPHS, bb.shape[0]), 0)
              ).astype(jnp.float32)
    pooled = jnp.dot(onehot, v, preferred_element_type=jnp.float32)

    @pl.when(pl.program_id(0) == 0)
    def _():
        o_ref[...] = jnp.zeros_like(o_ref)

    o_ref[...] += pooled


_BLK = 512
_GRID = NP // _BLK


def _mlp1(x_p, a0, a1, wa, ba, wb, bb):
    row = pl.BlockSpec((_BLK, HIDDEN), lambda i: (i, 0))
    full = pl.BlockSpec((HIDDEN, HIDDEN), lambda i: (0, 0))
    bias = pl.BlockSpec((1, HIDDEN), lambda i: (0, 0))
    return pl.pallas_call(
        _mlp1_body,
        grid=(_GRID,),
        in_specs=[row, row, row, full, bias, full, bias],
        out_specs=row,
        out_shape=jax.ShapeDtypeStruct((NP, HIDDEN), jnp.float32),
    )(x_p, a0, a1, wa, ba.reshape(1, HIDDEN), wb, bb.reshape(1, HIDDEN))


def _mlp2(h1, a0, a1, wa, ba, wb, bb, wfc, bfc, batch_p):
    row = pl.BlockSpec((_BLK, HIDDEN), lambda i: (i, 0))
    full = pl.BlockSpec((HIDDEN, HIDDEN), lambda i: (0, 0))
    bias = pl.BlockSpec((1, HIDDEN), lambda i: (0, 0))
    return pl.pallas_call(
        _mlp2_body,
        grid=(_GRID,),
        in_specs=[row, row, row, full, bias, full, bias,
                  pl.BlockSpec((HIDDEN, 1), lambda i: (0, 0)),
                  pl.BlockSpec((1, 1), lambda i: (0, 0)),
                  pl.BlockSpec((1, 1, _BLK), lambda i: (i, 0, 0))],
        out_specs=pl.BlockSpec((N_GRAPHS, 1), lambda i: (0, 0)),
        out_shape=jax.ShapeDtypeStruct((N_GRAPHS, 1), jnp.float32),
    )(h1, a0, a1, wa, ba.reshape(1, HIDDEN), wb, bb.reshape(1, HIDDEN),
      wfc, bfc.reshape(1, 1), batch_p.reshape(_GRID, 1, _BLK))


def kernel(x, edge_index, batch, W1a, b1a, W1b, b1b, W2a, b2a, W2b, b2b, Wfc, bfc):
    pad = NW * EPW - N_EDGES
    ar = jnp.arange(pad, dtype=jnp.int32)
    src_p = jnp.concatenate([edge_index[0], (ar * 997) % N_NODES]).reshape(NW, CH, CHUNK)
    dst_p = jnp.concatenate([edge_index[1], N_NODES + (ar % (NP - N_NODES))]
                            ).reshape(NW, CH, CHUNK)

    x_p = jnp.pad(x, ((0, NP - N_NODES), (0, 0)))
    batch_p = jnp.pad(batch, (0, NP - N_NODES), constant_values=N_GRAPHS)

    agg1 = _make_sc_agg(N_NODES)(x, src_p, dst_p)
    h1 = _mlp1(x_p, agg1[0], agg1[1], W1a, b1a, W1b, b1b)
    agg2 = _make_sc_agg(NP)(h1, src_p, dst_p)
    out = _mlp2(h1, agg2[0], agg2[1], W2a, b2a, W2b, b2b, Wfc, bfc, batch_p)
    return out


# speedup vs baseline: 11.8831x; 1.4509x over previous
import functools

import jax
import jax.numpy as jnp
from jax import lax
from jax.experimental import pallas as pl
from jax.experimental.pallas import tpu as pltpu
from jax.experimental.pallas import tpu_sc as plsc

N_NODES = 10000
N_EDGES = 320000
HIDDEN = 128
N_GRAPHS = 64

NC = 2
NS = 16
NW = NC * NS
CHUNK = 128
IBLK = 8
NIB = 10
CH = NIB * IBLK
EPW = CH * CHUNK
NP = 10240
ROWS_PER_TILE = NP // NS


def _make_sc_agg(n_rows: int):
    mesh = plsc.VectorSubcoreMesh(core_axis_name="c", subcore_axis_name="s")

    @functools.partial(
        pl.kernel,
        mesh=mesh,
        out_type=jax.ShapeDtypeStruct((NC, NP, HIDDEN), jnp.float32),
        scratch_types=[
            pltpu.VMEM((IBLK, CHUNK), jnp.int32),
            pltpu.VMEM((IBLK, CHUNK), jnp.int32),
            pltpu.VMEM((IBLK, CHUNK), jnp.int32),
            pltpu.VMEM((IBLK, CHUNK), jnp.int32),
            pltpu.VMEM((CHUNK, HIDDEN), jnp.float32),
            pltpu.VMEM((CHUNK, HIDDEN), jnp.float32),
            pltpu.VMEM_SHARED((NP, HIDDEN), jnp.float32),
            pltpu.SemaphoreType.DMA,
            pltpu.SemaphoreType.DMA,
            pltpu.SemaphoreType.DMA,
            pltpu.SemaphoreType.DMA,
        ],
    )
    def k(x_hbm, src_hbm, dst_hbm, out_hbm,
          sib0, sib1, dib0, dib1, r0, r1, acc, si0, si1, sr0, sr1):
        sib = (sib0, sib1)
        dib = (dib0, dib1)
        rows = (r0, r1)
        semi = (si0, si1)
        semr = (sr0, sr1)
        c = lax.axis_index("c")
        s = lax.axis_index("s")
        w = c * NS + s

        def stage_idx(bi, p):
            pltpu.async_copy(src_hbm.at[w, bi], sib[p], semi[p])
            pltpu.async_copy(dst_hbm.at[w, bi], dib[p], semi[p])

        def wait_idx(bi, p):
            pltpu.make_async_copy(src_hbm.at[w, bi], sib[p], semi[p]).wait()
            pltpu.make_async_copy(dst_hbm.at[w, bi], dib[p], semi[p]).wait()

        stage_idx(0, 0)

        def zrow(i, carry):
            for j in range(HIDDEN // 16):
                rows[0][i, pl.ds(j * 16, 16)] = jnp.zeros((16,), jnp.float32)
            return carry

        lax.fori_loop(0, CHUNK, zrow, 0)
        for kk in range(ROWS_PER_TILE // CHUNK):
            pltpu.sync_copy(rows[0], acc.at[pl.ds(s * ROWS_PER_TILE + kk * CHUNK, CHUNK)])

        wait_idx(0, 0)
        plsc.subcore_barrier()

        pltpu.async_copy(x_hbm.at[sib[0].at[0]], rows[0], semr[0])
        pltpu.async_copy(x_hbm.at[sib[0].at[1]], rows[1], semr[1])

        def process_block(i, p):
            stage_idx(i + 1, p ^ 1)
            for q in range(IBLK):
                b = q % 2
                if q == IBLK - 2:
                    wait_idx(i + 1, p ^ 1)
                pltpu.make_async_copy(x_hbm.at[sib[p].at[q]], rows[b], semr[b]).wait()
                pltpu.sync_copy(rows[b], acc.at[dib[p].at[q]], add=True)
                if q < IBLK - 2:
                    pltpu.async_copy(x_hbm.at[sib[p].at[q + 2]], rows[b], semr[b])
                else:
                    pltpu.async_copy(x_hbm.at[sib[p ^ 1].at[q + 2 - IBLK]], rows[b], semr[b])

        def body(i2, carry):
            process_block(2 * i2, 0)
            process_block(2 * i2 + 1, 1)
            return carry

        lax.fori_loop(0, NIB // 2, body, 0)
        pltpu.make_async_copy(x_hbm.at[sib[0].at[0]], rows[0], semr[0]).wait()
        pltpu.make_async_copy(x_hbm.at[sib[0].at[1]], rows[1], semr[1]).wait()
        plsc.subcore_barrier()

        pltpu.sync_copy(
            acc.at[pl.ds(s * ROWS_PER_TILE, ROWS_PER_TILE)],
            out_hbm.at[c, pl.ds(s * ROWS_PER_TILE, ROWS_PER_TILE)],
        )

    return k


_INV_SQRT2 = 0.7071067811865476


def _gelu(z):
    return 0.5 * z * (1.0 + lax.erf(z * _INV_SQRT2))


def _mlp1_body(x_ref, a0_ref, a1_ref, wa_ref, ba_ref, wb_ref, bb_ref, o_ref):
    t = x_ref[...] + a0_ref[...] + a1_ref[...]
    z = jnp.dot(t, wa_ref[...], preferred_element_type=jnp.float32) + ba_ref[...]
    z = _gelu(z)
    z = jnp.dot(z, wb_ref[...], preferred_element_type=jnp.float32) + bb_ref[...]
    o_ref[...] = _gelu(z)


def _mlp2_body(x_ref, a0_ref, a1_ref, wa_ref, ba_ref, wb_ref, bb_ref,
               wfc_ref, bfc_ref, b_ref, o_ref):
    t = x_ref[...] + a0_ref[...] + a1_ref[...]
    z = jnp.dot(t, wa_ref[...], preferred_element_type=jnp.float32) + ba_ref[...]
    z = _gelu(z)
    z = jnp.dot(z, wb_ref[...], preferred_element_type=jnp.float32) + bb_ref[...]
    z = _gelu(z)
    v = jnp.dot(z, wfc_ref[...], preferred_element_type=jnp.float32) + bfc_ref[...]
    bb = b_ref[0, 0, :]
    onehot = (bb[None, :] == lax.broadcasted_iota(jnp.int32, (N_GRAPHS, bb.shape[0]), 0)
              ).astype(jnp.float32)
    pooled = jnp.dot(onehot, v, preferred_element_type=jnp.float32)

    @pl.when(pl.program_id(0) == 0)
    def _():
        o_ref[...] = jnp.zeros_like(o_ref)

    o_ref[...] += pooled


_BLK = 512
_GRID = NP // _BLK


def _mlp1(x_p, a0, a1, wa, ba, wb, bb):
    row = pl.BlockSpec((_BLK, HIDDEN), lambda i: (i, 0))
    full = pl.BlockSpec((HIDDEN, HIDDEN), lambda i: (0, 0))
    bias = pl.BlockSpec((1, HIDDEN), lambda i: (0, 0))
    return pl.pallas_call(
        _mlp1_body,
        grid=(_GRID,),
        in_specs=[row, row, row, full, bias, full, bias],
        out_specs=row,
        out_shape=jax.ShapeDtypeStruct((NP, HIDDEN), jnp.float32),
    )(x_p, a0, a1, wa, ba.reshape(1, HIDDEN), wb, bb.reshape(1, HIDDEN))


def _mlp2(h1, a0, a1, wa, ba, wb, bb, wfc, bfc, batch_p):
    row = pl.BlockSpec((_BLK, HIDDEN), lambda i: (i, 0))
    full = pl.BlockSpec((HIDDEN, HIDDEN), lambda i: (0, 0))
    bias = pl.BlockSpec((1, HIDDEN), lambda i: (0, 0))
    return pl.pallas_call(
        _mlp2_body,
        grid=(_GRID,),
        in_specs=[row, row, row, full, bias, full, bias,
                  pl.BlockSpec((HIDDEN, 1), lambda i: (0, 0)),
                  pl.BlockSpec((1, 1), lambda i: (0, 0)),
                  pl.BlockSpec((1, 1, _BLK), lambda i: (i, 0, 0))],
        out_specs=pl.BlockSpec((N_GRAPHS, 1), lambda i: (0, 0)),
        out_shape=jax.ShapeDtypeStruct((N_GRAPHS, 1), jnp.float32),
    )(h1, a0, a1, wa, ba.reshape(1, HIDDEN), wb, bb.reshape(1, HIDDEN),
      wfc, bfc.reshape(1, 1), batch_p.reshape(_GRID, 1, _BLK))


def kernel(x, edge_index, batch, W1a, b1a, W1b, b1b, W2a, b2a, W2b, b2b, Wfc, bfc):
    pad = NW * EPW - N_EDGES
    ar = jnp.arange(pad, dtype=jnp.int32)
    src_p = jnp.concatenate([edge_index[0], (ar * 997) % N_NODES]
                            ).reshape(NW, NIB, IBLK, CHUNK)
    dst_p = jnp.concatenate([edge_index[1], N_NODES + (ar % (NP - N_NODES))]
                            ).reshape(NW, NIB, IBLK, CHUNK)
    tail = ((jnp.arange(NW * IBLK * CHUNK, dtype=jnp.int32) * 613) % N_NODES
            ).reshape(NW, 1, IBLK, CHUNK)
    src_p = jnp.concatenate([src_p, tail], axis=1)
    dst_p = jnp.concatenate([dst_p, tail], axis=1)

    x_p = jnp.pad(x, ((0, NP - N_NODES), (0, 0)))
    batch_p = jnp.pad(batch, (0, NP - N_NODES), constant_values=N_GRAPHS)

    agg1 = _make_sc_agg(N_NODES)(x, src_p, dst_p)
    h1 = _mlp1(x_p, agg1[0], agg1[1], W1a, b1a, W1b, b1b)
    agg2 = _make_sc_agg(NP)(h1, src_p, dst_p)
    out = _mlp2(h1, agg2[0], agg2[1], W2a, b2a, W2b, b2b, Wfc, bfc, batch_p)
    return out


# speedup vs baseline: 12.5986x; 1.0602x over previous
import functools

import jax
import jax.numpy as jnp
from jax import lax
from jax.experimental import pallas as pl
from jax.experimental.pallas import tpu as pltpu
from jax.experimental.pallas import tpu_sc as plsc

N_NODES = 10000
N_EDGES = 320000
HIDDEN = 128
N_GRAPHS = 64

NC = 2
NS = 16
NW = NC * NS
CHUNK = 64
IBLK = 16
NIB = 10
RING = 4
CH = NIB * IBLK
EPW = CH * CHUNK
NP = 10240
ROWS_PER_TILE = NP // NS


def _make_sc_agg(n_rows: int):
    mesh = plsc.VectorSubcoreMesh(core_axis_name="c", subcore_axis_name="s")

    @functools.partial(
        pl.kernel,
        mesh=mesh,
        out_type=jax.ShapeDtypeStruct((NC, NP, HIDDEN), jnp.float32),
        scratch_types=[
            pltpu.VMEM((IBLK, CHUNK), jnp.int32),
            pltpu.VMEM((IBLK, CHUNK), jnp.int32),
            pltpu.VMEM((IBLK, CHUNK), jnp.int32),
            pltpu.VMEM((IBLK, CHUNK), jnp.int32),
        ] + [pltpu.VMEM((CHUNK, HIDDEN), jnp.float32) for _ in range(RING)]
          + [pltpu.VMEM_SHARED((NP, HIDDEN), jnp.float32)]
          + [pltpu.SemaphoreType.DMA for _ in range(2 + RING)],
    )
    def k(x_hbm, src_hbm, dst_hbm, out_hbm, sib0, sib1, dib0, dib1, *rest):
        sib = (sib0, sib1)
        dib = (dib0, dib1)
        rows = rest[:RING]
        acc = rest[RING]
        semi = rest[RING + 1: RING + 3]
        semr = rest[RING + 3:]
        c = lax.axis_index("c")
        s = lax.axis_index("s")
        w = c * NS + s

        def stage_idx(bi, p):
            pltpu.async_copy(src_hbm.at[w, bi], sib[p], semi[p])
            pltpu.async_copy(dst_hbm.at[w, bi], dib[p], semi[p])

        def wait_idx(bi, p):
            pltpu.make_async_copy(src_hbm.at[w, bi], sib[p], semi[p]).wait()
            pltpu.make_async_copy(dst_hbm.at[w, bi], dib[p], semi[p]).wait()

        stage_idx(0, 0)

        def zrow(i, carry):
            for j in range(HIDDEN // 16):
                rows[0][i, pl.ds(j * 16, 16)] = jnp.zeros((16,), jnp.float32)
            return carry

        lax.fori_loop(0, CHUNK, zrow, 0)
        for kk in range(ROWS_PER_TILE // CHUNK):
            pltpu.sync_copy(rows[0], acc.at[pl.ds(s * ROWS_PER_TILE + kk * CHUNK, CHUNK)])

        wait_idx(0, 0)
        plsc.subcore_barrier()

        for b in range(RING):
            pltpu.async_copy(x_hbm.at[sib[0].at[b]], rows[b], semr[b])

        def process_block(i, p):
            stage_idx(i + 1, p ^ 1)
            for q in range(IBLK):
                b = q % RING
                if q == IBLK - RING:
                    wait_idx(i + 1, p ^ 1)
                pltpu.make_async_copy(x_hbm.at[sib[p].at[q]], rows[b], semr[b]).wait()
                pltpu.sync_copy(rows[b], acc.at[dib[p].at[q]], add=True)
                if q < IBLK - RING:
                    pltpu.async_copy(x_hbm.at[sib[p].at[q + RING]], rows[b], semr[b])
                else:
                    pltpu.async_copy(x_hbm.at[sib[p ^ 1].at[q + RING - IBLK]], rows[b], semr[b])

        def body(i2, carry):
            process_block(2 * i2, 0)
            process_block(2 * i2 + 1, 1)
            return carry

        lax.fori_loop(0, NIB // 2, body, 0)
        for b in range(RING):
            pltpu.make_async_copy(x_hbm.at[sib[0].at[b]], rows[b], semr[b]).wait()
        plsc.subcore_barrier()

        pltpu.sync_copy(
            acc.at[pl.ds(s * ROWS_PER_TILE, ROWS_PER_TILE)],
            out_hbm.at[c, pl.ds(s * ROWS_PER_TILE, ROWS_PER_TILE)],
        )

    return k


_INV_SQRT2 = 0.7071067811865476


def _gelu(z):
    return 0.5 * z * (1.0 + lax.erf(z * _INV_SQRT2))


def _mlp1_body(x_ref, a0_ref, a1_ref, wa_ref, ba_ref, wb_ref, bb_ref, o_ref):
    t = x_ref[...] + a0_ref[...] + a1_ref[...]
    z = jnp.dot(t, wa_ref[...], preferred_element_type=jnp.float32) + ba_ref[...]
    z = _gelu(z)
    z = jnp.dot(z, wb_ref[...], preferred_element_type=jnp.float32) + bb_ref[...]
    o_ref[...] = _gelu(z)


def _mlp2_body(x_ref, a0_ref, a1_ref, wa_ref, ba_ref, wb_ref, bb_ref,
               wfc_ref, bfc_ref, b_ref, o_ref):
    t = x_ref[...] + a0_ref[...] + a1_ref[...]
    z = jnp.dot(t, wa_ref[...], preferred_element_type=jnp.float32) + ba_ref[...]
    z = _gelu(z)
    z = jnp.dot(z, wb_ref[...], preferred_element_type=jnp.float32) + bb_ref[...]
    z = _gelu(z)
    v = jnp.dot(z, wfc_ref[...], preferred_element_type=jnp.float32) + bfc_ref[...]
    bb = b_ref[0, 0, :]
    onehot = (bb[None, :] == lax.broadcasted_iota(jnp.int32, (N_GRAPHS, bb.shape[0]), 0)
              ).astype(jnp.float32)
    pooled = jnp.dot(onehot, v, preferred_element_type=jnp.float32)

    @pl.when(pl.program_id(0) == 0)
    def _():
        o_ref[...] = jnp.zeros_like(o_ref)

    o_ref[...] += pooled


_BLK = 512
_GRID = NP // _BLK


def _mlp1(x_p, a0, a1, wa, ba, wb, bb):
    row = pl.BlockSpec((_BLK, HIDDEN), lambda i: (i, 0))
    full = pl.BlockSpec((HIDDEN, HIDDEN), lambda i: (0, 0))
    bias = pl.BlockSpec((1, HIDDEN), lambda i: (0, 0))
    return pl.pallas_call(
        _mlp1_body,
        grid=(_GRID,),
        in_specs=[row, row, row, full, bias, full, bias],
        out_specs=row,
        out_shape=jax.ShapeDtypeStruct((NP, HIDDEN), jnp.float32),
    )(x_p, a0, a1, wa, ba.reshape(1, HIDDEN), wb, bb.reshape(1, HIDDEN))


def _mlp2(h1, a0, a1, wa, ba, wb, bb, wfc, bfc, batch_p):
    row = pl.BlockSpec((_BLK, HIDDEN), lambda i: (i, 0))
    full = pl.BlockSpec((HIDDEN, HIDDEN), lambda i: (0, 0))
    bias = pl.BlockSpec((1, HIDDEN), lambda i: (0, 0))
    return pl.pallas_call(
        _mlp2_body,
        grid=(_GRID,),
        in_specs=[row, row, row, full, bias, full, bias,
                  pl.BlockSpec((HIDDEN, 1), lambda i: (0, 0)),
                  pl.BlockSpec((1, 1), lambda i: (0, 0)),
                  pl.BlockSpec((1, 1, _BLK), lambda i: (i, 0, 0))],
        out_specs=pl.BlockSpec((N_GRAPHS, 1), lambda i: (0, 0)),
        out_shape=jax.ShapeDtypeStruct((N_GRAPHS, 1), jnp.float32),
    )(h1, a0, a1, wa, ba.reshape(1, HIDDEN), wb, bb.reshape(1, HIDDEN),
      wfc, bfc.reshape(1, 1), batch_p.reshape(_GRID, 1, _BLK))


def kernel(x, edge_index, batch, W1a, b1a, W1b, b1b, W2a, b2a, W2b, b2b, Wfc, bfc):
    pad = NW * EPW - N_EDGES
    ar = jnp.arange(pad, dtype=jnp.int32)
    src_p = jnp.concatenate([edge_index[0], (ar * 997) % N_NODES]
                            ).reshape(NW, NIB, IBLK, CHUNK)
    dst_p = jnp.concatenate([edge_index[1], N_NODES + (ar % (NP - N_NODES))]
                            ).reshape(NW, NIB, IBLK, CHUNK)
    tail = ((jnp.arange(NW * IBLK * CHUNK, dtype=jnp.int32) * 613) % N_NODES
            ).reshape(NW, 1, IBLK, CHUNK)
    src_p = jnp.concatenate([src_p, tail], axis=1)
    dst_p = jnp.concatenate([dst_p, tail], axis=1)

    x_p = jnp.pad(x, ((0, NP - N_NODES), (0, 0)))
    batch_p = jnp.pad(batch, (0, NP - N_NODES), constant_values=N_GRAPHS)

    agg1 = _make_sc_agg(N_NODES)(x, src_p, dst_p)
    h1 = _mlp1(x_p, agg1[0], agg1[1], W1a, b1a, W1b, b1b)
    agg2 = _make_sc_agg(NP)(h1, src_p, dst_p)
    out = _mlp2(h1, agg2[0], agg2[1], W2a, b2a, W2b, b2b, Wfc, bfc, batch_p)
    return out


# speedup vs baseline: 13.2997x; 1.0556x over previous
import functools

import jax
import jax.numpy as jnp
import numpy as np
from jax import lax
from jax.experimental import pallas as pl
from jax.experimental.pallas import tpu as pltpu
from jax.experimental.pallas import tpu_sc as plsc

N_NODES = 10000
N_EDGES = 320000
HIDDEN = 128
N_GRAPHS = 64

NC = 2
NS = 16
NW = NC * NS
CHUNK = 64
IBLK = 16
NIB = 10
RING = 4
CH = NIB * IBLK
EPW = CH * CHUNK
NP = 10240
ROWS_PER_TILE = NP // NS


def _make_sc_agg(n_rows: int):
    mesh = plsc.VectorSubcoreMesh(core_axis_name="c", subcore_axis_name="s")

    @functools.partial(
        pl.kernel,
        mesh=mesh,
        out_type=jax.ShapeDtypeStruct((NC, NP, HIDDEN), jnp.float32),
        scratch_types=[
            pltpu.VMEM((IBLK, CHUNK), jnp.int32),
            pltpu.VMEM((IBLK, CHUNK), jnp.int32),
            pltpu.VMEM((IBLK, CHUNK), jnp.int32),
            pltpu.VMEM((IBLK, CHUNK), jnp.int32),
        ] + [pltpu.VMEM((CHUNK, HIDDEN), jnp.float32) for _ in range(RING)]
          + [pltpu.VMEM_SHARED((NP, HIDDEN), jnp.float32)]
          + [pltpu.SemaphoreType.DMA for _ in range(2 + RING)],
    )
    def k(x_hbm, src_hbm, dst_hbm, out_hbm, sib0, sib1, dib0, dib1, *rest):
        sib = (sib0, sib1)
        dib = (dib0, dib1)
        rows = rest[:RING]
        acc = rest[RING]
        semi = rest[RING + 1: RING + 3]
        semr = rest[RING + 3:]
        c = lax.axis_index("c")
        s = lax.axis_index("s")
        w = c * NS + s

        def stage_idx(bi, p):
            pltpu.async_copy(src_hbm.at[w, bi], sib[p], semi[p])
            pltpu.async_copy(dst_hbm.at[w, bi], dib[p], semi[p])

        def wait_idx(bi, p):
            pltpu.make_async_copy(src_hbm.at[w, bi], sib[p], semi[p]).wait()
            pltpu.make_async_copy(dst_hbm.at[w, bi], dib[p], semi[p]).wait()

        stage_idx(0, 0)

        def zrow(i, carry):
            for j in range(HIDDEN // 16):
                rows[0][i, pl.ds(j * 16, 16)] = jnp.zeros((16,), jnp.float32)
            return carry

        lax.fori_loop(0, CHUNK, zrow, 0)
        for kk in range(ROWS_PER_TILE // CHUNK):
            pltpu.sync_copy(rows[0], acc.at[pl.ds(s * ROWS_PER_TILE + kk * CHUNK, CHUNK)])

        wait_idx(0, 0)
        plsc.subcore_barrier()

        for b in range(RING):
            pltpu.async_copy(x_hbm.at[sib[0].at[b]], rows[b], semr[b])

        def process_block(i, p):
            stage_idx(i + 1, p ^ 1)
            for q in range(IBLK):
                b = q % RING
                if q == IBLK - RING:
                    wait_idx(i + 1, p ^ 1)
                pltpu.make_async_copy(x_hbm.at[sib[p].at[q]], rows[b], semr[b]).wait()
                pltpu.sync_copy(rows[b], acc.at[dib[p].at[q]], add=True)
                if q < IBLK - RING:
                    pltpu.async_copy(x_hbm.at[sib[p].at[q + RING]], rows[b], semr[b])
                else:
                    pltpu.async_copy(x_hbm.at[sib[p ^ 1].at[q + RING - IBLK]], rows[b], semr[b])

        def body(i2, carry):
            process_block(2 * i2, 0)
            process_block(2 * i2 + 1, 1)
            return carry

        lax.fori_loop(0, NIB // 2, body, 0)
        for b in range(RING):
            pltpu.make_async_copy(x_hbm.at[sib[0].at[b]], rows[b], semr[b]).wait()
        plsc.subcore_barrier()

        pltpu.sync_copy(
            acc.at[pl.ds(s * ROWS_PER_TILE, ROWS_PER_TILE)],
            out_hbm.at[c, pl.ds(s * ROWS_PER_TILE, ROWS_PER_TILE)],
        )

    return k


_INV_SQRT2 = 0.7071067811865476


def _gelu(z):
    return 0.5 * z * (1.0 + lax.erf(z * _INV_SQRT2))


def _mlp1_body(x_ref, a0_ref, a1_ref, wa_ref, ba_ref, wb_ref, bb_ref, o_ref):
    t = x_ref[...] + a0_ref[0] + a1_ref[0]
    z = jnp.dot(t, wa_ref[...], preferred_element_type=jnp.float32) + ba_ref[...]
    z = _gelu(z)
    z = jnp.dot(z, wb_ref[...], preferred_element_type=jnp.float32) + bb_ref[...]
    o_ref[...] = _gelu(z)


def _mlp2_body(x_ref, a0_ref, a1_ref, wa_ref, ba_ref, wb_ref, bb_ref,
               wfc_ref, bfc_ref, b_ref, o_ref):
    t = x_ref[...] + a0_ref[0] + a1_ref[0]
    z = jnp.dot(t, wa_ref[...], preferred_element_type=jnp.float32) + ba_ref[...]
    z = _gelu(z)
    z = jnp.dot(z, wb_ref[...], preferred_element_type=jnp.float32) + bb_ref[...]
    z = _gelu(z)
    v = jnp.dot(z, wfc_ref[...], preferred_element_type=jnp.float32) + bfc_ref[...]
    bb = b_ref[0, 0, :]
    onehot = (bb[None, :] == lax.broadcasted_iota(jnp.int32, (N_GRAPHS, bb.shape[0]), 0)
              ).astype(jnp.float32)
    pooled = jnp.dot(onehot, v, preferred_element_type=jnp.float32)

    @pl.when(pl.program_id(0) == 0)
    def _():
        o_ref[...] = jnp.zeros_like(o_ref)

    o_ref[...] += pooled


_BLK = 512
_GRID = NP // _BLK


_ROW = pl.BlockSpec((_BLK, HIDDEN), lambda i: (i, 0))
_AGG0 = pl.BlockSpec((1, _BLK, HIDDEN), lambda i: (0, i, 0))
_AGG1 = pl.BlockSpec((1, _BLK, HIDDEN), lambda i: (1, i, 0))
_FULL = pl.BlockSpec((HIDDEN, HIDDEN), lambda i: (0, 0))
_BIAS = pl.BlockSpec((1, HIDDEN), lambda i: (0, 0))


def _mlp1(x_p, agg, wa, ba, wb, bb):
    return pl.pallas_call(
        _mlp1_body,
        grid=(_GRID,),
        in_specs=[_ROW, _AGG0, _AGG1, _FULL, _BIAS, _FULL, _BIAS],
        out_specs=_ROW,
        out_shape=jax.ShapeDtypeStruct((NP, HIDDEN), jnp.float32),
    )(x_p, agg, agg, wa, ba.reshape(1, HIDDEN), wb, bb.reshape(1, HIDDEN))


def _mlp2(h1, agg, wa, ba, wb, bb, wfc, bfc, batch_p):
    return pl.pallas_call(
        _mlp2_body,
        grid=(_GRID,),
        in_specs=[_ROW, _AGG0, _AGG1, _FULL, _BIAS, _FULL, _BIAS,
                  pl.BlockSpec((HIDDEN, 1), lambda i: (0, 0)),
                  pl.BlockSpec((1, 1), lambda i: (0, 0)),
                  pl.BlockSpec((1, 1, _BLK), lambda i: (i, 0, 0))],
        out_specs=pl.BlockSpec((N_GRAPHS, 1), lambda i: (0, 0)),
        out_shape=jax.ShapeDtypeStruct((N_GRAPHS, 1), jnp.float32),
    )(h1, agg, agg, wa, ba.reshape(1, HIDDEN), wb, bb.reshape(1, HIDDEN),
      wfc, bfc.reshape(1, 1), batch_p.reshape(_GRID, 1, _BLK))


def kernel(x, edge_index, batch, W1a, b1a, W1b, b1b, W2a, b2a, W2b, b2b, Wfc, bfc):
    pad = NW * EPW - N_EDGES
    ar = np.arange(pad, dtype=np.int32)
    src_pad = jnp.asarray((ar * 997) % N_NODES)
    dst_pad = jnp.asarray(N_NODES + (ar % (NP - N_NODES)))
    tail = jnp.asarray(
        ((np.arange(NW * IBLK * CHUNK, dtype=np.int32) * 613) % N_NODES
         ).reshape(NW, 1, IBLK, CHUNK))
    src_p = jnp.concatenate(
        [jnp.concatenate([edge_index[0], src_pad]).reshape(NW, NIB, IBLK, CHUNK),
         tail], axis=1)
    dst_p = jnp.concatenate(
        [jnp.concatenate([edge_index[1], dst_pad]).reshape(NW, NIB, IBLK, CHUNK),
         tail], axis=1)

    x_p = jnp.pad(x, ((0, NP - N_NODES), (0, 0)))
    batch_p = jnp.pad(batch, (0, NP - N_NODES), constant_values=N_GRAPHS)

    agg1 = _make_sc_agg(N_NODES)(x, src_p, dst_p)
    h1 = _mlp1(x_p, agg1, W1a, b1a, W1b, b1b)
    agg2 = _make_sc_agg(NP)(h1, src_p, dst_p)
    out = _mlp2(h1, agg2, W2a, b2a, W2b, b2b, Wfc, bfc, batch_p)
    return out
